# Initial kernel scaffold; baseline (speedup 1.0000x reference)
import jax
import jax.numpy as jnp
from jax.experimental import pallas as pl


def kernel(input_ids, attention_mask, embedding_dict, input_ids2dict_ids):
    raise NotImplementedError("write your pallas kernel here")


# speedup vs baseline: 10.2269x; 10.2269x over previous
import functools

import jax
import jax.numpy as jnp
from jax import lax
from jax.experimental import pallas as pl
from jax.experimental.pallas import tpu as pltpu
from jax.experimental.pallas import tpu_sc as plsc

VOCAB = 100000
EMBED_DIM = 64
BATCH = 4096
HIST = 50
N_IDS = BATCH * HIST

NC, NS = 2, 16
NW = NC * NS
PER_W = N_IDS // NW
CHUNK = 128
N_CHUNKS = PER_W // CHUNK


_MESH = plsc.VectorSubcoreMesh(core_axis_name="c", subcore_axis_name="s")


@functools.partial(
    pl.kernel,
    out_type=jax.ShapeDtypeStruct((N_IDS, EMBED_DIM), jnp.float32),
    mesh=_MESH,
    compiler_params=pltpu.CompilerParams(use_tc_tiling_on_sc=False),
    scratch_types=[
        pltpu.VMEM((N_CHUNKS, CHUNK), jnp.int32),
        pltpu.VMEM((N_CHUNKS, CHUNK), jnp.int32),
        pltpu.VMEM((2, CHUNK, EMBED_DIM), jnp.float32),
        pltpu.SemaphoreType.DMA,
        pltpu.SemaphoreType.DMA,
        pltpu.SemaphoreType.DMA,
    ],
)
def _gather_kernel(ids_hbm, remap_hbm, table_hbm, out_hbm,
                   ids_v, idx2_v, rows_v, sem_ids, sem_remap, sem_rows):
    wid = lax.axis_index("s") * NC + lax.axis_index("c")
    base = wid * PER_W

    pltpu.sync_copy(ids_hbm.at[wid], ids_v)

    def fire_remap(j, carry):
        pltpu.async_copy(remap_hbm.at[ids_v.at[j]], idx2_v.at[j], sem_remap)
        return carry

    lax.fori_loop(0, N_CHUNKS, fire_remap, 0)
    pltpu.make_async_copy(ids_hbm.at[wid], idx2_v, sem_remap).wait()

    pltpu.async_copy(table_hbm.at[idx2_v.at[0]], rows_v.at[0], sem_rows)

    def step(j, carry):
        slot = lax.rem(j, 2)
        nxt = lax.rem(j + 1, 2)

        @pl.when(j + 1 < N_CHUNKS)
        def _():
            pltpu.async_copy(table_hbm.at[idx2_v.at[j + 1]], rows_v.at[nxt],
                             sem_rows)

        pltpu.make_async_copy(table_hbm.at[idx2_v.at[0]], rows_v.at[slot],
                              sem_rows).wait()
        pltpu.sync_copy(rows_v.at[slot],
                        out_hbm.at[pl.ds(base + j * CHUNK, CHUNK)])
        return carry

    lax.fori_loop(0, N_CHUNKS, step, 0)


def kernel(input_ids, attention_mask, embedding_dict, input_ids2dict_ids):
    ids = input_ids.reshape(NW, N_CHUNKS, CHUNK)
    flat = _gather_kernel(ids, input_ids2dict_ids, embedding_dict)
    return flat.reshape(BATCH, HIST, EMBED_DIM), attention_mask


# speedup vs baseline: 10.5290x; 1.0295x over previous
import functools

import jax
import jax.numpy as jnp
from jax import lax
from jax.experimental import pallas as pl
from jax.experimental.pallas import tpu as pltpu
from jax.experimental.pallas import tpu_sc as plsc

VOCAB = 100000
EMBED_DIM = 64
BATCH = 4096
HIST = 50
N_IDS = BATCH * HIST

NC, NS = 2, 16
NW = NC * NS
PER_W = N_IDS // NW
CHUNK = 128
N_CHUNKS = PER_W // CHUNK
RING = 5
GROUPS = N_CHUNKS // RING

_MESH = plsc.VectorSubcoreMesh(core_axis_name="c", subcore_axis_name="s")


@functools.partial(
    pl.kernel,
    out_type=jax.ShapeDtypeStruct((N_IDS, EMBED_DIM), jnp.float32),
    mesh=_MESH,
    compiler_params=pltpu.CompilerParams(use_tc_tiling_on_sc=False),
    scratch_types=[
        pltpu.VMEM((N_CHUNKS, CHUNK), jnp.int32),
        pltpu.VMEM((N_CHUNKS, CHUNK), jnp.int32),
        pltpu.VMEM((RING, CHUNK, EMBED_DIM), jnp.float32),
        pltpu.SemaphoreType.DMA,
        [pltpu.SemaphoreType.DMA] * RING,
        [pltpu.SemaphoreType.DMA] * RING,
    ],
)
def _gather_kernel(ids_hbm, remap_hbm, table_hbm, out_hbm,
                   ids_v, idx2_v, rows_v, sem_remap, sems_g, sems_wb):
    wid = lax.axis_index("s") * NC + lax.axis_index("c")
    base = wid * PER_W

    pltpu.sync_copy(ids_hbm.at[wid], ids_v)

    def fire_remap(j, carry):
        pltpu.async_copy(remap_hbm.at[ids_v.at[j]], idx2_v.at[j], sem_remap)
        return carry

    lax.fori_loop(0, N_CHUNKS, fire_remap, 0)
    pltpu.make_async_copy(ids_hbm.at[wid], idx2_v, sem_remap).wait()

    for b in range(RING):
        pltpu.async_copy(table_hbm.at[idx2_v.at[b]], rows_v.at[b], sems_g[b])

    def group(g, carry):
        for b in range(RING):
            j = g * RING + b
            pltpu.make_async_copy(table_hbm.at[idx2_v.at[0]], rows_v.at[b],
                                  sems_g[b]).wait()
            pltpu.async_copy(rows_v.at[b],
                             out_hbm.at[pl.ds(base + j * CHUNK, CHUNK)],
                             sems_wb[b])

            @pl.when(j + RING < N_CHUNKS)
            def _():
                pltpu.make_async_copy(rows_v.at[b],
                                      out_hbm.at[pl.ds(base, CHUNK)],
                                      sems_wb[b]).wait()
                pltpu.async_copy(table_hbm.at[idx2_v.at[j + RING]],
                                 rows_v.at[b], sems_g[b])
        return carry

    lax.fori_loop(0, GROUPS, group, 0)

    for b in range(RING):
        pltpu.make_async_copy(rows_v.at[b], out_hbm.at[pl.ds(base, CHUNK)],
                              sems_wb[b]).wait()


def kernel(input_ids, attention_mask, embedding_dict, input_ids2dict_ids):
    ids = input_ids.reshape(NW, N_CHUNKS, CHUNK)
    flat = _gather_kernel(ids, input_ids2dict_ids, embedding_dict)
    return flat.reshape(BATCH, HIST, EMBED_DIM), attention_mask
